# baseline (speedup 1.0000x reference)
import functools

import jax
import jax.numpy as jnp
from jax import lax
from jax.experimental import pallas as pl
from jax.experimental.pallas import tpu as pltpu
from jax.experimental.pallas import tpu_sc as plsc

N = 10000
E = 320000
F = 128
NQ = 4
FQ = F // NQ
EF = 16
NEG_SLOPE = 0.2

NC = 2
NS = 16
CH = 128
E_PAD = ((E + NS * CH - 1) // (NS * CH)) * (NS * CH)
EPW = E_PAD // NS
NCH = EPW // CH
while NCH % 4:
    E_PAD += NS * CH
    EPW = E_PAD // NS
    NCH = EPW // CH
NPAD = 10240
RPT = NPAD // NS

BN = 1000
GRID = N // BN
BE = E // GRID


def _dense_body(x_ref, w_ref, al_ref, ar_ref, eft_ref, we_ref, ae_ref,
                f0_ref, f1_ref, f2_ref, f3_ref, el_ref, er_ref, ee_ref):
    f = jnp.dot(x_ref[...], w_ref[...], preferred_element_type=jnp.float32)
    for q, fq_ref in enumerate((f0_ref, f1_ref, f2_ref, f3_ref)):
        fq_ref[...] = f[:, q * FQ:(q + 1) * FQ]
    el_ref[0, 0, :] = jnp.sum(f * al_ref[0, :][None, :], axis=1)
    er_ref[0, 0, :] = jnp.sum(f * ar_ref[0, :][None, :], axis=1)
    v = jnp.sum(we_ref[...] * ae_ref[0, :][None, :], axis=1)
    ee_ref[0, 0, :] = jnp.sum(eft_ref[...] * v[:, None], axis=0)


_dense = pl.pallas_call(
    _dense_body,
    grid=(GRID,),
    in_specs=[
        pl.BlockSpec((BN, F), lambda i: (i, 0)),
        pl.BlockSpec((F, F), lambda i: (0, 0)),
        pl.BlockSpec((1, F), lambda i: (0, 0)),
        pl.BlockSpec((1, F), lambda i: (0, 0)),
        pl.BlockSpec((EF, BE), lambda i: (0, i)),
        pl.BlockSpec((EF, F), lambda i: (0, 0)),
        pl.BlockSpec((1, F), lambda i: (0, 0)),
    ],
    out_specs=[pl.BlockSpec((BN, FQ), lambda i: (i, 0))] * NQ + [
        pl.BlockSpec((1, 1, BN), lambda i: (i, 0, 0)),
        pl.BlockSpec((1, 1, BN), lambda i: (i, 0, 0)),
        pl.BlockSpec((1, 1, BE), lambda i: (i, 0, 0)),
    ],
    out_shape=[jax.ShapeDtypeStruct((N, FQ), jnp.float32)] * NQ + [
        jax.ShapeDtypeStruct((GRID, 1, BN), jnp.float32),
        jax.ShapeDtypeStruct((GRID, 1, BN), jnp.float32),
        jax.ShapeDtypeStruct((GRID, 1, BE), jnp.float32),
    ],
)


_sc_mesh = plsc.VectorSubcoreMesh(core_axis_name="c", subcore_axis_name="s")


@functools.partial(
    pl.kernel,
    out_type=[
        jax.ShapeDtypeStruct((NQ, NPAD, FQ), jnp.float32),
        jax.ShapeDtypeStruct((NPAD,), jnp.float32),
    ],
    mesh=_sc_mesh,
    compiler_params=pltpu.CompilerParams(needs_layout_passes=False,
                                         use_tc_tiling_on_sc=False),
    scratch_types=[
        pltpu.VMEM((NPAD,), jnp.float32),
        pltpu.VMEM((NPAD,), jnp.float32),
        pltpu.VMEM((EPW,), jnp.int32),
        pltpu.VMEM((NCH, CH), jnp.int32),
        pltpu.VMEM((EPW,), jnp.float32),
        pltpu.VMEM((CH, FQ), jnp.float32),
        pltpu.VMEM((CH, FQ), jnp.float32),
        pltpu.VMEM((CH, FQ), jnp.float32),
        pltpu.VMEM((CH, FQ), jnp.float32),
        pltpu.VMEM_SHARED((NPAD, FQ), jnp.float32),
        pltpu.VMEM_SHARED((NPAD,), jnp.float32),
        pltpu.SemaphoreType.DMA,
        pltpu.SemaphoreType.DMA,
        pltpu.SemaphoreType.DMA,
        pltpu.SemaphoreType.DMA,
    ],
)
def _sc_aggregate(f0_hbm, f1_hbm, f2_hbm, f3_hbm, src_hbm, dst2_hbm,
                  el_hbm, er_hbm, ee_hbm,
                  u_out, esum_out,
                  el_v, er_v, src_v, dst2_v, ee_v, rows0, rows1,
                  srows0, srows1, u_sh, esum_sh, gsem0, gsem1, ssem0, ssem1):
    cid = lax.axis_index("c")
    sid = lax.axis_index("s")
    ebase = pl.multiple_of(sid * EPW, CH)
    rbase = pl.multiple_of(sid * RPT, CH)

    zeros16 = jnp.zeros((16,), jnp.float32)

    def _zero_row(r, carry):
        for q in range(FQ // 16):
            rows0[r, pl.ds(q * 16, 16)] = zeros16
        return carry

    lax.fori_loop(0, CH, _zero_row, 0)

    def _zero_e(i, carry):
        ee_v[pl.ds(i * 16, 16)] = zeros16
        return carry

    lax.fori_loop(0, RPT // 16, _zero_e, 0)

    for k in range(RPT // CH):
        pltpu.sync_copy(rows0, u_sh.at[pl.ds(rbase + k * CH, CH)])
    pltpu.sync_copy(ee_v.at[pl.ds(0, RPT)], esum_sh.at[pl.ds(rbase, RPT)])

    pltpu.sync_copy(el_hbm, el_v.at[pl.ds(0, N)])
    pltpu.sync_copy(er_hbm, er_v.at[pl.ds(0, N)])
    pltpu.sync_copy(src_hbm.at[pl.ds(ebase, EPW)], src_v)
    pltpu.sync_copy(dst2_hbm.at[sid], dst2_v)
    pltpu.sync_copy(ee_hbm.at[pl.ds(ebase, EPW)], ee_v)

    def _ex_body(i, carry):
        sl = pl.ds(i * 16, 16)
        lg = (plsc.load_gather(el_v, [src_v[sl]])
              + plsc.load_gather(er_v, [dst2_v[i // 8, pl.ds((i % 8) * 16, 16)]])
              + ee_v[sl])
        lg = jnp.where(lg >= 0, lg, NEG_SLOPE * lg)
        ee_v[sl] = jnp.exp(lg)
        return carry

    lax.fori_loop(0, EPW // 16, _ex_body, 0, unroll=2)
    plsc.subcore_barrier()

    def _gather(feat_hbm, c, rows, sem):
        pltpu.async_copy(feat_hbm.at[src_v.at[pl.ds(c * CH, CH)]], rows, sem)

    def _wait_gather(feat_hbm, c, rows, sem):
        pltpu.make_async_copy(feat_hbm.at[src_v.at[pl.ds(c * CH, CH)]],
                              rows, sem).wait()

    def _process(c, rows, do_esum):
        base = c * CH

        def _scale(j, carry):
            av = plsc.load_gather(ee_v, [jnp.full((16,), base + j, jnp.int32)])
            for q in range(FQ // 16):
                sl = pl.ds(q * 16, 16)
                rows[j, sl] = rows[j, sl] * av
            return carry

        lax.fori_loop(0, CH, _scale, 0, unroll=2)
        pltpu.sync_copy(rows, u_sh.at[dst2_v.at[c]], add=True)
        if do_esum:
            pltpu.sync_copy(ee_v.at[pl.ds(base, CH)], esum_sh.at[dst2_v.at[c]],
                            add=True)

    def _pass(feat_hbm, qid, do_esum, zero_next):
        ring = (rows0, rows1, srows0, srows1)
        sems = (gsem0, gsem1, ssem0, ssem1)
        for k in range(3):
            _gather(feat_hbm, k, ring[k], sems[k])

        def _outer(g, carry):
            c0 = g * 4
            for k in range(4):
                c = c0 + k
                kn = (k + 3) % 4

                @pl.when(c + 3 < NCH)
                def _():
                    _gather(feat_hbm, c + 3, ring[kn], sems[kn])

                _wait_gather(feat_hbm, c, ring[k], sems[k])
                _process(c, ring[k], do_esum)
            return carry

        lax.fori_loop(0, NCH // 4, _outer, 0)
        plsc.subcore_barrier()
        pltpu.sync_copy(u_sh.at[pl.ds(rbase, RPT)],
                        u_out.at[qid, pl.ds(rbase, RPT)])
        if do_esum:
            pltpu.sync_copy(esum_sh.at[pl.ds(rbase, RPT)],
                            esum_out.at[pl.ds(rbase, RPT)])
        if zero_next:
            def _zr1(r, carry):
                for q in range(FQ // 16):
                    rows1[r, pl.ds(q * 16, 16)] = zeros16
                return carry

            lax.fori_loop(0, CH, _zr1, 0)
            for k in range(RPT // CH):
                pltpu.sync_copy(rows1, u_sh.at[pl.ds(rbase + k * CH, CH)])
            plsc.subcore_barrier()

    @pl.when(cid == 0)
    def _():
        _pass(f0_hbm, 0, True, True)
        _pass(f1_hbm, 1, False, False)

    @pl.when(cid == 1)
    def _():
        _pass(f2_hbm, 2, False, True)
        _pass(f3_hbm, 3, False, False)


def _final_body(u_ref, es_ref, x_ref, b_ref, out_ref):
    u = jnp.concatenate([u_ref[q] for q in range(NQ)], axis=1)
    inv = 1.0 / (es_ref[:, 0] + 1e-9)
    out_ref[...] = u * inv[:, None] + x_ref[...] + b_ref[0, :][None, :]


_final = pl.pallas_call(
    _final_body,
    grid=(GRID,),
    in_specs=[
        pl.BlockSpec((NQ, BN, FQ), lambda i: (0, i, 0)),
        pl.BlockSpec((BN, 1), lambda i: (i, 0)),
        pl.BlockSpec((BN, F), lambda i: (i, 0)),
        pl.BlockSpec((1, F), lambda i: (0, 0)),
    ],
    out_specs=pl.BlockSpec((BN, F), lambda i: (i, 0)),
    out_shape=jax.ShapeDtypeStruct((N, F), jnp.float32),
)


def kernel(node_features, edge_index, edge_features, W, W_edge,
           attn_l, attn_r, attn_e, bias):
    src = edge_index[0]
    dst = edge_index[1]

    f0, f1, f2, f3, el2, er2, ee2 = _dense(
        node_features, W, attn_l.reshape(1, F), attn_r.reshape(1, F),
        edge_features.T, W_edge, attn_e.reshape(1, F))
    el = el2.reshape(N)
    er = er2.reshape(N)
    ee = ee2.reshape(E)

    pad = E_PAD - E
    spread = (jnp.arange(pad, dtype=jnp.int32) * 97) % N
    src_p = jnp.concatenate([src, spread])
    dst_p = jnp.concatenate([dst, spread])
    ee_p = jnp.concatenate([ee, jnp.full((pad,), -1e30, jnp.float32)])
    dst2 = dst_p.reshape(NS, NCH, CH)

    u, esum = _sc_aggregate(f0, f1, f2, f3, src_p, dst2, el, er, ee_p)
    return _final(u, esum.reshape(NPAD, 1), node_features,
                  bias.reshape(1, F))


# speedup vs baseline: 1.0122x; 1.0122x over previous
import functools

import jax
import jax.numpy as jnp
from jax import lax
from jax.experimental import pallas as pl
from jax.experimental.pallas import tpu as pltpu
from jax.experimental.pallas import tpu_sc as plsc

N = 10000
E = 320000
F = 128
NQ = 4
FQ = F // NQ
EF = 16
NEG_SLOPE = 0.2

NC = 2
NS = 16
CH = 128
E_PAD = ((E + NS * CH - 1) // (NS * CH)) * (NS * CH)
EPW = E_PAD // NS
NCH = EPW // CH
if NCH % 2:
    E_PAD += NS * CH
    EPW = E_PAD // NS
    NCH = EPW // CH
NPAD = 10240
RPT = NPAD // NS

BN = 1000
GRID = N // BN
BE = E // GRID


def _dense_body(x_ref, w_ref, al_ref, ar_ref, eft_ref, we_ref, ae_ref,
                f0_ref, f1_ref, f2_ref, f3_ref, el_ref, er_ref, ee_ref):
    f = jnp.dot(x_ref[...], w_ref[...], preferred_element_type=jnp.float32)
    for q, fq_ref in enumerate((f0_ref, f1_ref, f2_ref, f3_ref)):
        fq_ref[...] = f[:, q * FQ:(q + 1) * FQ]
    el_ref[0, 0, :] = jnp.sum(f * al_ref[0, :][None, :], axis=1)
    er_ref[0, 0, :] = jnp.sum(f * ar_ref[0, :][None, :], axis=1)
    v = jnp.sum(we_ref[...] * ae_ref[0, :][None, :], axis=1)
    ee_ref[0, 0, :] = jnp.sum(eft_ref[...] * v[:, None], axis=0)


_dense = pl.pallas_call(
    _dense_body,
    grid=(GRID,),
    in_specs=[
        pl.BlockSpec((BN, F), lambda i: (i, 0)),
        pl.BlockSpec((F, F), lambda i: (0, 0)),
        pl.BlockSpec((1, F), lambda i: (0, 0)),
        pl.BlockSpec((1, F), lambda i: (0, 0)),
        pl.BlockSpec((EF, BE), lambda i: (0, i)),
        pl.BlockSpec((EF, F), lambda i: (0, 0)),
        pl.BlockSpec((1, F), lambda i: (0, 0)),
    ],
    out_specs=[pl.BlockSpec((BN, FQ), lambda i: (i, 0))] * NQ + [
        pl.BlockSpec((1, 1, BN), lambda i: (i, 0, 0)),
        pl.BlockSpec((1, 1, BN), lambda i: (i, 0, 0)),
        pl.BlockSpec((1, 1, BE), lambda i: (i, 0, 0)),
    ],
    out_shape=[jax.ShapeDtypeStruct((N, FQ), jnp.float32)] * NQ + [
        jax.ShapeDtypeStruct((GRID, 1, BN), jnp.float32),
        jax.ShapeDtypeStruct((GRID, 1, BN), jnp.float32),
        jax.ShapeDtypeStruct((GRID, 1, BE), jnp.float32),
    ],
)


_sc_mesh = plsc.VectorSubcoreMesh(core_axis_name="c", subcore_axis_name="s")


@functools.partial(
    pl.kernel,
    out_type=[
        jax.ShapeDtypeStruct((NQ, NPAD, FQ), jnp.float32),
        jax.ShapeDtypeStruct((NPAD,), jnp.float32),
    ],
    mesh=_sc_mesh,
    compiler_params=pltpu.CompilerParams(needs_layout_passes=False,
                                         use_tc_tiling_on_sc=False),
    scratch_types=[
        pltpu.VMEM((NPAD,), jnp.float32),
        pltpu.VMEM((NPAD,), jnp.float32),
        pltpu.VMEM((EPW,), jnp.int32),
        pltpu.VMEM((NCH, CH), jnp.int32),
        pltpu.VMEM((EPW,), jnp.float32),
        pltpu.VMEM((CH, FQ), jnp.float32),
        pltpu.VMEM((CH, FQ), jnp.float32),
        pltpu.VMEM((RPT // 4, FQ), jnp.float32),
        pltpu.VMEM_SHARED((NPAD, FQ), jnp.float32),
        pltpu.VMEM_SHARED((NPAD,), jnp.float32),
        pltpu.SemaphoreType.DMA,
        pltpu.SemaphoreType.DMA,
    ],
)
def _sc_aggregate(f0_hbm, f1_hbm, f2_hbm, f3_hbm, src_hbm, dst2_hbm,
                  el_hbm, er_hbm, ee_hbm,
                  u_out, esum_out,
                  el_v, er_v, src_v, dst2_v, ee_v, rows0, rows1,
                  zbuf, u_sh, esum_sh, gsem0, gsem1):
    cid = lax.axis_index("c")
    sid = lax.axis_index("s")
    ebase = pl.multiple_of(sid * EPW, CH)
    rbase = pl.multiple_of(sid * RPT, CH)

    zeros16 = jnp.zeros((16,), jnp.float32)
    ZR = RPT // 4

    def _zero_row(r, carry):
        for q in range(FQ // 16):
            zbuf[r, pl.ds(q * 16, 16)] = zeros16
        return carry

    lax.fori_loop(0, ZR, _zero_row, 0)

    def _zero_e(i, carry):
        ee_v[pl.ds(i * 16, 16)] = zeros16
        return carry

    lax.fori_loop(0, RPT // 16, _zero_e, 0)

    for k in range(4):
        pltpu.sync_copy(zbuf, u_sh.at[pl.ds(rbase + k * ZR, ZR)])
    pltpu.sync_copy(ee_v.at[pl.ds(0, RPT)], esum_sh.at[pl.ds(rbase, RPT)])

    pltpu.sync_copy(el_hbm, el_v.at[pl.ds(0, N)])
    pltpu.sync_copy(er_hbm, er_v.at[pl.ds(0, N)])
    pltpu.sync_copy(src_hbm.at[pl.ds(ebase, EPW)], src_v)
    pltpu.sync_copy(dst2_hbm.at[sid], dst2_v)
    pltpu.sync_copy(ee_hbm.at[pl.ds(ebase, EPW)], ee_v)

    def _ex_body(i, carry):
        sl = pl.ds(i * 16, 16)
        lg = (plsc.load_gather(el_v, [src_v[sl]])
              + plsc.load_gather(er_v, [dst2_v[i // 8, pl.ds((i % 8) * 16, 16)]])
              + ee_v[sl])
        lg = jnp.where(lg >= 0, lg, NEG_SLOPE * lg)
        ee_v[sl] = jnp.exp(lg)
        return carry

    lax.fori_loop(0, EPW // 16, _ex_body, 0, unroll=2)
    plsc.subcore_barrier()

    def _gather(feat_hbm, c, rows, sem):
        pltpu.async_copy(feat_hbm.at[src_v.at[pl.ds(c * CH, CH)]], rows, sem)

    def _wait_gather(feat_hbm, c, rows, sem):
        pltpu.make_async_copy(feat_hbm.at[src_v.at[pl.ds(c * CH, CH)]],
                              rows, sem).wait()

    def _process(c, rows, do_esum):
        base = c * CH

        def _scale(j, carry):
            av = plsc.load_gather(ee_v, [jnp.full((16,), base + j, jnp.int32)])
            for q in range(FQ // 16):
                sl = pl.ds(q * 16, 16)
                rows[j, sl] = rows[j, sl] * av
            return carry

        lax.fori_loop(0, CH, _scale, 0, unroll=2)
        pltpu.sync_copy(rows, u_sh.at[dst2_v.at[c]], add=True)
        if do_esum:
            pltpu.sync_copy(ee_v.at[pl.ds(base, CH)], esum_sh.at[dst2_v.at[c]],
                            add=True)

    def _pass(feat_hbm, qid, do_esum, zero_next):
        _gather(feat_hbm, 0, rows0, gsem0)

        def _outer(g, carry):
            c0 = g * 2
            c1 = c0 + 1
            _wait_gather(feat_hbm, c0, rows0, gsem0)
            _gather(feat_hbm, c1, rows1, gsem1)
            _process(c0, rows0, do_esum)
            _wait_gather(feat_hbm, c1, rows1, gsem1)

            @pl.when(c1 + 1 < NCH)
            def _():
                _gather(feat_hbm, c1 + 1, rows0, gsem0)

            _process(c1, rows1, do_esum)
            return carry

        lax.fori_loop(0, NCH // 2, _outer, 0)
        plsc.subcore_barrier()
        pltpu.sync_copy(u_sh.at[pl.ds(rbase, RPT)],
                        u_out.at[qid, pl.ds(rbase, RPT)])
        if do_esum:
            pltpu.sync_copy(esum_sh.at[pl.ds(rbase, RPT)],
                            esum_out.at[pl.ds(rbase, RPT)])
        if zero_next:
            for k in range(4):
                pltpu.sync_copy(zbuf, u_sh.at[pl.ds(rbase + k * ZR, ZR)])
            plsc.subcore_barrier()

    @pl.when(cid == 0)
    def _():
        _pass(f0_hbm, 0, True, True)
        _pass(f1_hbm, 1, False, False)

    @pl.when(cid == 1)
    def _():
        _pass(f2_hbm, 2, False, True)
        _pass(f3_hbm, 3, False, False)


def _final_body(u_ref, es_ref, x_ref, b_ref, out_ref):
    u = jnp.concatenate([u_ref[q] for q in range(NQ)], axis=1)
    inv = 1.0 / (es_ref[:, 0] + 1e-9)
    out_ref[...] = u * inv[:, None] + x_ref[...] + b_ref[0, :][None, :]


_final = pl.pallas_call(
    _final_body,
    grid=(GRID,),
    in_specs=[
        pl.BlockSpec((NQ, BN, FQ), lambda i: (0, i, 0)),
        pl.BlockSpec((BN, 1), lambda i: (i, 0)),
        pl.BlockSpec((BN, F), lambda i: (i, 0)),
        pl.BlockSpec((1, F), lambda i: (0, 0)),
    ],
    out_specs=pl.BlockSpec((BN, F), lambda i: (i, 0)),
    out_shape=jax.ShapeDtypeStruct((N, F), jnp.float32),
)


def kernel(node_features, edge_index, edge_features, W, W_edge,
           attn_l, attn_r, attn_e, bias):
    src = edge_index[0]
    dst = edge_index[1]

    f0, f1, f2, f3, el2, er2, ee2 = _dense(
        node_features, W, attn_l.reshape(1, F), attn_r.reshape(1, F),
        edge_features.T, W_edge, attn_e.reshape(1, F))
    el = el2.reshape(N)
    er = er2.reshape(N)
    ee = ee2.reshape(E)

    pad = E_PAD - E
    spread = (jnp.arange(pad, dtype=jnp.int32) * 97) % N
    src_p = jnp.concatenate([src, spread])
    dst_p = jnp.concatenate([dst, spread])
    ee_p = jnp.concatenate([ee, jnp.full((pad,), -1e30, jnp.float32)])
    dst2 = dst_p.reshape(NS, NCH, CH)

    u, esum = _sc_aggregate(f0, f1, f2, f3, src_p, dst2, el, er, ee_p)
    return _final(u, esum.reshape(NPAD, 1), node_features,
                  bias.reshape(1, F))
